# baseline (speedup 1.0000x reference)
import functools

import jax
import jax.numpy as jnp
from jax import lax
from jax.experimental import pallas as pl
from jax.experimental.pallas import tpu as pltpu
from jax.experimental.pallas import tpu_sc as plsc

N = 10000
E = 320000
G = 64
N_PAD = 10240
NEG_INF = float("-inf")

_MESH = plsc.VectorSubcoreMesh(core_axis_name="c", subcore_axis_name="s")
_NTILES = 16
_NCORES = 2


def _fill(ref, n, val, dtype):
    def body(i, _):
        ref[pl.ds(i * 16, 16)] = jnp.full((16,), val, dtype)
        return 0
    lax.fori_loop(0, n // 16, body, 0)


_DEG_K = 80
_EPW = E // (_NCORES * _NTILES)


@functools.partial(
    pl.kernel,
    out_type=jax.ShapeDtypeStruct((_NCORES * N_PAD,), jnp.float32),
    mesh=_MESH,
    scratch_types=[
        pltpu.VMEM((_EPW,), jnp.int32),
        pltpu.VMEM((_DEG_K,), jnp.int32),
        pltpu.VMEM((_DEG_K,), jnp.float32),
        pltpu.VMEM((N_PAD // _NTILES,), jnp.float32),
        pltpu.VMEM_SHARED((N_PAD,), jnp.float32),
    ],
)
def _deg_kernel(dstg_hbm, out_hbm, dst_v, idx_v, ones_v, z_v, acc_sh):
    c = lax.axis_index("c")
    s = lax.axis_index("s")
    w = c * _NTILES + s
    stripe = N_PAD // _NTILES

    _fill(z_v, stripe, 0.0, jnp.float32)
    _fill(ones_v, _DEG_K, 1.0, jnp.float32)
    pltpu.sync_copy(z_v, acc_sh.at[pl.ds(s * stripe, stripe)])
    pltpu.sync_copy(dstg_hbm.at[pl.ds(w * _EPW, _EPW)], dst_v)
    plsc.subcore_barrier()

    def chunk(j, _):
        for t in range(_DEG_K // 16):
            idx_v[pl.ds(t * 16, 16)] = dst_v[pl.ds(j * _DEG_K + t * 16, 16)]
        pltpu.sync_copy(ones_v, acc_sh.at[idx_v], add=True)
        return 0
    lax.fori_loop(0, _EPW // _DEG_K, chunk, 0)

    plsc.subcore_barrier()
    pltpu.sync_copy(acc_sh.at[pl.ds(s * stripe, stripe)],
                    out_hbm.at[pl.ds(c * N_PAD + s * stripe, stripe)])


def _make_agg(d_feat, n_ranges):
    rpr = N_PAD // n_ranges
    stripe = rpr // _NTILES
    ept = E // _NTILES
    wedges = 800
    nwin = ept // wedges
    krows = 64

    @functools.partial(
        pl.kernel,
        out_type=jax.ShapeDtypeStruct((N_PAD, d_feat), jnp.float32),
        mesh=_MESH,
        compiler_params=pltpu.CompilerParams(needs_layout_passes=False),
        scratch_types=[
            pltpu.VMEM((wedges,), jnp.int32),
            pltpu.VMEM((wedges,), jnp.int32),
            pltpu.VMEM((128,), jnp.int32),
            pltpu.VMEM((krows,), jnp.int32),
            pltpu.VMEM((krows,), jnp.int32),
            pltpu.VMEM((krows, d_feat), jnp.float32),
            pltpu.VMEM_SHARED((rpr + 16, d_feat), jnp.float32),
            pltpu.SemaphoreType.DMA,
        ],
    )
    def agg(table_hbm, srcg_hbm, dstg_hbm, zeros_hbm, out_hbm,
            sw, dw, ring, sbuf, dbuf, gbuf, acc_sh, sem):
        c = lax.axis_index("c")
        s = lax.axis_index("s")
        lane = lax.iota(jnp.int32, 16)
        base = s * ept

        def flush():
            for t in range(krows // 16):
                v = ring[pl.ds(t * 16, 16)]
                sbuf[pl.ds(t * 16, 16)] = v & 0x3FFF
                dbuf[pl.ds(t * 16, 16)] = v >> 14
            pltpu.async_copy(table_hbm.at[sbuf], gbuf, sem).wait()
            for t in range(krows // 16):
                iv = dbuf[pl.ds(t * 16, 16)]
                pltpu.sync_copy(gbuf.at[pl.ds(t * 16, 16)],
                                acc_sh.at[iv], add=True)

        for p in range(n_ranges // _NCORES):
            r = c * (n_ranges // _NCORES) + p
            lo = r * rpr

            pltpu.sync_copy(zeros_hbm, acc_sh.at[pl.ds(s * stripe, stripe)])

            @pl.when(s == 0)
            def _():
                pltpu.sync_copy(zeros_hbm.at[pl.ds(0, 16)],
                                acc_sh.at[pl.ds(rpr, 16)])
            plsc.subcore_barrier()

            def grp(i, pn):
                sv = sw[pl.ds(i * 16, 16)]
                dv = dw[pl.ds(i * 16, 16)]
                m = (dv >= lo) & (dv < lo + rpr)
                packed = sv | ((dv - lo) << 14)
                ps = plsc.cumsum(m.astype(jnp.int32))
                pos = pn + ps - 1
                plsc.store_scatter(ring, [pos], packed, mask=m)
                pn = pn + ps[15]

                @pl.when(pn >= krows)
                def _():
                    flush()
                    ring[pl.ds(0, 16)] = ring[pl.ds(krows, 16)]
                return jnp.where(pn >= krows, pn - krows, pn)

            def win(w, pn):
                pltpu.sync_copy(srcg_hbm.at[pl.ds(base + w * wedges, wedges)],
                                sw)
                pltpu.sync_copy(dstg_hbm.at[pl.ds(base + w * wedges, wedges)],
                                dw)
                return lax.fori_loop(0, wedges // 16, grp, pn)

            pn = lax.fori_loop(0, nwin, win, jnp.int32(0))

            dummy = jnp.full((16,), rpr << 14, jnp.int32)
            for t in range(krows // 16):
                plsc.store_scatter(ring, [pn + t * 16 + lane], dummy)
            flush()

            plsc.subcore_barrier()
            for q in range(stripe // 32):
                pltpu.sync_copy(
                    acc_sh.at[pl.ds(s * stripe + q * 32, 32)],
                    out_hbm.at[pl.ds(lo + s * stripe + q * 32, 32)])
            plsc.subcore_barrier()

    return agg


_agg512 = _make_agg(512, 4)
_agg256 = _make_agg(256, 2)


_BM = 1000


def _tc_stage1(x_blk, dga_blk, dgb_blk, w1_blk, h1_ref, h1s_ref):
    deg = dga_blk[...] + dgb_blk[...] + 1.0
    dinv = lax.rsqrt(deg)
    h1 = jnp.dot(x_blk[...], w1_blk[...], preferred_element_type=jnp.float32)
    h1_ref[...] = h1
    h1s_ref[...] = h1 * dinv


def _tc_stage2(agg_blk, h1_blk, dga_blk, dgb_blk, w2_blk, b1_blk,
               h2_ref, h2s_ref):
    deg = dga_blk[...] + dgb_blk[...] + 1.0
    dinv = lax.rsqrt(deg)
    v = dinv * agg_blk[...] + h1_blk[...] / deg + b1_blk[...]
    u = jnp.where(v >= 0, v, 0.01 * v)
    h2 = jnp.dot(u, w2_blk[...], preferred_element_type=jnp.float32)
    h2_ref[...] = h2
    h2s_ref[...] = h2 * dinv


def _tc_stage3(agg_blk, h2_blk, dga_blk, dgb_blk, b2_blk, batch_blk,
               fw1_blk, fb1_blk, fw2_blk, fb2_blk, fw3_blk, fb3_blk,
               out_ref, embs_ref):
    i = pl.program_id(0)

    @pl.when(i == 0)
    def _():
        out_ref[...] = jnp.full(out_ref.shape, NEG_INF, jnp.float32)
        embs_ref[...] = jnp.full(embs_ref.shape, NEG_INF, jnp.float32)

    deg = dga_blk[...] + dgb_blk[...] + 1.0
    dinv = lax.rsqrt(deg)
    h = dinv * agg_blk[...] + h2_blk[...] / deg + b2_blk[...]
    y = jnp.dot(h, fw1_blk[...], preferred_element_type=jnp.float32) + fb1_blk[...]
    y = jnp.dot(y, fw2_blk[...], preferred_element_type=jnp.float32) + fb2_blk[...]
    y = jnp.dot(y, fw3_blk[...], preferred_element_type=jnp.float32) + fb3_blk[...]

    b = batch_blk[...]
    for g in range(G):
        m = b == g
        hm = jnp.max(jnp.where(m, h, NEG_INF), axis=0, keepdims=True)
        ym = jnp.max(jnp.where(m, y, NEG_INF), axis=0, keepdims=True)
        embs_ref[g:g + 1, :] = jnp.maximum(embs_ref[g:g + 1, :], hm)
        out_ref[g:g + 1, :] = jnp.maximum(out_ref[g:g + 1, :], ym)


def _unused_kernel(x, edge_index, batch, W1, b1, W2, b2, fW1, fb1, fW2, fb2, fW3, fb3):
    f32 = jnp.float32
    nblk = N // _BM

    srcg = edge_index[0]
    dstg = edge_index[1]
    degs = _deg_kernel(dstg)
    dga = degs[:N_PAD].reshape(N_PAD, 1)
    dgb = degs[N_PAD:].reshape(N_PAD, 1)

    col = lambda bm: pl.BlockSpec((bm, 1), lambda i: (i, 0))
    full = lambda a, b: pl.BlockSpec((a, b), lambda i: (0, 0))

    h1, h1s = pl.pallas_call(
        _tc_stage1,
        grid=(nblk,),
        in_specs=[
            pl.BlockSpec((_BM, 128), lambda i: (i, 0)),
            col(_BM), col(_BM),
            full(128, 512),
        ],
        out_specs=[pl.BlockSpec((_BM, 512), lambda i: (i, 0))] * 2,
        out_shape=[jax.ShapeDtypeStruct((N, 512), f32)] * 2,
    )(x, dga[:N], dgb[:N], W1)

    z512 = jnp.zeros((N_PAD // 4 // _NTILES, 512), f32)
    z256 = jnp.zeros((N_PAD // 2 // _NTILES, 256), f32)
    agg1 = _agg512(h1s, srcg, dstg, z512)

    h2, h2s = pl.pallas_call(
        _tc_stage2,
        grid=(nblk,),
        in_specs=[
            pl.BlockSpec((_BM, 512), lambda i: (i, 0)),
            pl.BlockSpec((_BM, 512), lambda i: (i, 0)),
            col(_BM), col(_BM),
            full(512, 256), full(1, 512),
        ],
        out_specs=[pl.BlockSpec((_BM, 256), lambda i: (i, 0))] * 2,
        out_shape=[jax.ShapeDtypeStruct((N, 256), f32)] * 2,
    )(agg1, h1, dga[:N], dgb[:N], W2, b1.reshape(1, 512))

    agg2 = _agg256(h2s, srcg, dstg, z256)

    out, embs = pl.pallas_call(
        _tc_stage3,
        grid=(nblk,),
        in_specs=[
            pl.BlockSpec((_BM, 256), lambda i: (i, 0)),
            pl.BlockSpec((_BM, 256), lambda i: (i, 0)),
            col(_BM), col(_BM),
            full(1, 256), col(_BM),
            full(256, 128), full(1, 128), full(128, 32), full(1, 32),
            full(32, 10), full(1, 10),
        ],
        out_specs=[
            pl.BlockSpec((G, 10), lambda i: (0, 0)),
            pl.BlockSpec((G, 256), lambda i: (0, 0)),
        ],
        out_shape=[
            jax.ShapeDtypeStruct((G, 10), f32),
            jax.ShapeDtypeStruct((G, 256), f32),
        ],
    )(agg2, h2, dga[:N], dgb[:N], b2.reshape(1, 256), batch.reshape(N, 1),
      fW1, fb1.reshape(1, 128), fW2, fb2.reshape(1, 32),
      fW3, fb3.reshape(1, 10))

    return (out, embs)


def kernel(x, edge_index, batch, W1, b1, W2, b2, fW1, fb1, fW2, fb2, fW3, fb3):
    def conv(h, W, b):
        src = edge_index[0]
        dst = edge_index[1]
        loop = jnp.arange(h.shape[0], dtype=src.dtype)
        src2 = jnp.concatenate([src, loop])
        dst2 = jnp.concatenate([dst, loop])
        deg = jnp.zeros((h.shape[0],), h.dtype).at[dst2].add(1.0)
        dinv = jnp.where(deg > 0, jax.lax.rsqrt(jnp.maximum(deg, 1e-12)), 0.0)
        norm = dinv[src2] * dinv[dst2]
        hw = h @ W
        msg = hw[src2] * norm[:, None]
        out = jnp.zeros((h.shape[0], W.shape[1]), h.dtype).at[dst2].add(msg)
        return out + b
    h = conv(x, W1, b1)
    h = jnp.where(h >= 0, h, 0.01 * h)
    h = conv(h, W2, b2)
    embs = jax.ops.segment_max(h, batch, num_segments=G)
    y = h @ fW1 + fb1
    y = y @ fW2 + fb2
    y = y @ fW3 + fb3
    out = jax.ops.segment_max(y, batch, num_segments=G)
    return (out, embs)


# speedup vs baseline: 2.7984x; 2.7984x over previous
import functools

import jax
import jax.numpy as jnp
from jax import lax
from jax.experimental import pallas as pl
from jax.experimental.pallas import tpu as pltpu
from jax.experimental.pallas import tpu_sc as plsc

N = 10000
E = 320000
G = 64
N_PAD = 10240
N_PAD2 = 11520
NEG_INF = float("-inf")

_MESH = plsc.VectorSubcoreMesh(core_axis_name="c", subcore_axis_name="s")
_NTILES = 16
_NCORES = 2
_NW = _NCORES * _NTILES


def _fill(ref, n, val, dtype):
    def body(i, _):
        ref[pl.ds(i * 16, 16)] = jnp.full((16,), val, dtype)
        return 0
    lax.fori_loop(0, n // 16, body, 0)


_DEG_K = 80
_EPW = E // _NW


@functools.partial(
    pl.kernel,
    out_type=jax.ShapeDtypeStruct((_NCORES * N_PAD,), jnp.float32),
    mesh=_MESH,
    scratch_types=[
        pltpu.VMEM((_EPW,), jnp.int32),
        pltpu.VMEM((_DEG_K,), jnp.int32),
        pltpu.VMEM((_DEG_K,), jnp.float32),
        pltpu.VMEM((N_PAD // _NTILES,), jnp.float32),
        pltpu.VMEM_SHARED((N_PAD,), jnp.float32),
    ],
)
def _deg_kernel(dstg_hbm, out_hbm, dst_v, idx_v, ones_v, z_v, acc_sh):
    c = lax.axis_index("c")
    s = lax.axis_index("s")
    w = c * _NTILES + s
    stripe = N_PAD // _NTILES

    _fill(z_v, stripe, 0.0, jnp.float32)
    _fill(ones_v, _DEG_K, 1.0, jnp.float32)
    pltpu.sync_copy(z_v, acc_sh.at[pl.ds(s * stripe, stripe)])
    pltpu.sync_copy(dstg_hbm.at[pl.ds(w * _EPW, _EPW)], dst_v)
    plsc.subcore_barrier()

    def chunk(j, _):
        for t in range(_DEG_K // 16):
            idx_v[pl.ds(t * 16, 16)] = dst_v[pl.ds(j * _DEG_K + t * 16, 16)]
        pltpu.sync_copy(ones_v, acc_sh.at[idx_v], add=True)
        return 0
    lax.fori_loop(0, _EPW // _DEG_K, chunk, 0)

    plsc.subcore_barrier()
    pltpu.sync_copy(acc_sh.at[pl.ds(s * stripe, stripe)],
                    out_hbm.at[pl.ds(c * N_PAD + s * stripe, stripe)])


_WE = 1600


def _make_agg(d_feat, rng_rows, n_pass, krows):
    nwin = E // _WE
    accw = (rng_rows + 1) * d_feat

    @functools.partial(
        pl.kernel,
        out_type=jax.ShapeDtypeStruct((N_PAD2 * d_feat,), jnp.float32),
        mesh=_MESH,
        compiler_params=pltpu.CompilerParams(needs_layout_passes=False),
        scratch_types=[
            pltpu.VMEM((2 * _WE,), jnp.int32),
            pltpu.VMEM((2 * _WE,), jnp.int32),
            pltpu.VMEM((128,), jnp.int32),
            pltpu.VMEM((krows * d_feat,), jnp.float32),
            pltpu.VMEM((accw,), jnp.float32),
            pltpu.SemaphoreType.DMA,
            pltpu.SemaphoreType.DMA,
            pltpu.SemaphoreType.DMA,
        ],
    )
    def agg(table_hbm, srcg_hbm, dstg_hbm, zeros_hbm, out_hbm,
            sw, dw, ring, gbuf, acc, semg, sema, semb):
        c = lax.axis_index("c")
        s = lax.axis_index("s")
        w = c * _NTILES + s
        lane = lax.iota(jnp.int32, 16)

        def start_win(i):
            off = (i % 2) * _WE
            sl = pl.ds(i * _WE, _WE)

            @pl.when(i % 2 == 0)
            def _():
                pltpu.async_copy(srcg_hbm.at[sl], sw.at[pl.ds(off, _WE)],
                                 sema)
                pltpu.async_copy(dstg_hbm.at[sl], dw.at[pl.ds(off, _WE)],
                                 sema)

            @pl.when(i % 2 == 1)
            def _():
                pltpu.async_copy(srcg_hbm.at[sl], sw.at[pl.ds(off, _WE)],
                                 semb)
                pltpu.async_copy(dstg_hbm.at[sl], dw.at[pl.ds(off, _WE)],
                                 semb)

        def wait_win(i):
            off = (i % 2) * _WE
            sl = pl.ds(i * _WE, _WE)

            @pl.when(i % 2 == 0)
            def _():
                pltpu.make_async_copy(srcg_hbm.at[sl],
                                      sw.at[pl.ds(off, _WE)], sema).wait()
                pltpu.make_async_copy(dstg_hbm.at[sl],
                                      dw.at[pl.ds(off, _WE)], sema).wait()

            @pl.when(i % 2 == 1)
            def _():
                pltpu.make_async_copy(srcg_hbm.at[sl],
                                      sw.at[pl.ds(off, _WE)], semb).wait()
                pltpu.make_async_copy(dstg_hbm.at[sl],
                                      dw.at[pl.ds(off, _WE)], semb).wait()

        def flush():
            v = ring[pl.ds(0, 16)]
            siv = v & 0x3FFF
            dlv = v >> 14
            for l in range(16):
                si = siv[l]
                pltpu.async_copy(
                    table_hbm.at[pl.ds(si * d_feat, d_feat)],
                    gbuf.at[pl.ds(l * d_feat, d_feat)], semg)
            for l in range(16):
                si = siv[l]
                pltpu.make_async_copy(
                    table_hbm.at[pl.ds(si * d_feat, d_feat)],
                    gbuf.at[pl.ds(l * d_feat, d_feat)], semg).wait()
            for l in range(16):
                off = dlv[l] * d_feat
                for q in range(d_feat // 16):
                    plsc.addupdate(acc.at[pl.ds(off + q * 16, 16)],
                                   gbuf[pl.ds(l * d_feat + q * 16, 16)])

        for p in range(n_pass):
            lo = (p * _NW + w) * rng_rows

            pltpu.sync_copy(zeros_hbm, acc)

            def grp(i, pn):
                sv = sw[pl.ds(i * 16, 16)]
                dv = dw[pl.ds(i * 16, 16)]
                m = (dv >= lo) & (dv < lo + rng_rows)
                packed = sv | ((dv - lo) << 14)
                ps = plsc.cumsum(m.astype(jnp.int32))
                pos = pn + ps - 1
                plsc.store_scatter(ring, [pos], packed, mask=m)
                pn = pn + ps[15]

                @pl.when(pn >= krows)
                def _():
                    flush()
                    ring[pl.ds(0, 16)] = ring[pl.ds(krows, 16)]
                return jnp.where(pn >= krows, pn - krows, pn)

            def win_body(i, pn):
                wait_win(i)

                @pl.when(i + 1 < nwin)
                def _():
                    start_win(i + 1)
                base = (i % 2) * (_WE // 16)

                def grp2(k, pn):
                    return grp(base + k, pn)
                return lax.fori_loop(0, _WE // 16, grp2, pn)

            start_win(0)
            pn = lax.fori_loop(0, nwin, win_body, jnp.int32(0))

            dummy = jnp.full((16,), rng_rows << 14, jnp.int32)
            for t in range(krows // 16):
                plsc.store_scatter(ring, [pn + t * 16 + lane], dummy)
            flush()

            pltpu.sync_copy(acc.at[pl.ds(0, rng_rows * d_feat)],
                            out_hbm.at[pl.ds(lo * d_feat,
                                             rng_rows * d_feat)])

    return agg


_agg512 = _make_agg(512, 180, 2, 16)
_agg256 = _make_agg(256, 360, 1, 16)


_BM = 1000


def _tc_stage1(x_blk, dga_blk, dgb_blk, w1_blk, h1_ref, h1s_ref):
    deg = dga_blk[...] + dgb_blk[...] + 1.0
    dinv = lax.rsqrt(deg)
    h1 = jnp.dot(x_blk[...], w1_blk[...], preferred_element_type=jnp.float32)
    h1_ref[...] = h1
    h1s_ref[...] = h1 * dinv


def _tc_stage2(agg_blk, h1_blk, dga_blk, dgb_blk, w2_blk, b1_blk,
               h2_ref, h2s_ref):
    deg = dga_blk[...] + dgb_blk[...] + 1.0
    dinv = lax.rsqrt(deg)
    v = dinv * agg_blk[...] + h1_blk[...] / deg + b1_blk[...]
    u = jnp.where(v >= 0, v, 0.01 * v)
    h2 = jnp.dot(u, w2_blk[...], preferred_element_type=jnp.float32)
    h2_ref[...] = h2
    h2s_ref[...] = h2 * dinv


def _tc_stage3(agg_blk, h2_blk, dga_blk, dgb_blk, b2_blk, batch_blk,
               fw1_blk, fb1_blk, fw2_blk, fb2_blk, fw3_blk, fb3_blk,
               out_ref, embs_ref):
    i = pl.program_id(0)

    @pl.when(i == 0)
    def _():
        out_ref[...] = jnp.full(out_ref.shape, NEG_INF, jnp.float32)
        embs_ref[...] = jnp.full(embs_ref.shape, NEG_INF, jnp.float32)

    deg = dga_blk[...] + dgb_blk[...] + 1.0
    dinv = lax.rsqrt(deg)
    h = dinv * agg_blk[...] + h2_blk[...] / deg + b2_blk[...]
    y = jnp.dot(h, fw1_blk[...],
                preferred_element_type=jnp.float32) + fb1_blk[...]
    y = jnp.dot(y, fw2_blk[...],
                preferred_element_type=jnp.float32) + fb2_blk[...]
    y = jnp.dot(y, fw3_blk[...],
                preferred_element_type=jnp.float32) + fb3_blk[...]

    b = batch_blk[...]
    for g in range(G):
        m = b == g
        hm = jnp.max(jnp.where(m, h, NEG_INF), axis=0, keepdims=True)
        ym = jnp.max(jnp.where(m, y, NEG_INF), axis=0, keepdims=True)
        embs_ref[g:g + 1, :] = jnp.maximum(embs_ref[g:g + 1, :], hm)
        out_ref[g:g + 1, :] = jnp.maximum(out_ref[g:g + 1, :], ym)


def kernel(x, edge_index, batch, W1, b1, W2, b2, fW1, fb1, fW2, fb2, fW3, fb3):
    f32 = jnp.float32
    nblk = N // _BM

    srcg = edge_index[0]
    dstg = edge_index[1]
    degs = _deg_kernel(dstg)
    dga = degs[:N].reshape(N, 1)
    dgb = degs[N_PAD:N_PAD + N].reshape(N, 1)

    col = lambda bm: pl.BlockSpec((bm, 1), lambda i: (i, 0))
    full = lambda a, b: pl.BlockSpec((a, b), lambda i: (0, 0))

    h1, h1s = pl.pallas_call(
        _tc_stage1,
        grid=(nblk,),
        in_specs=[
            pl.BlockSpec((_BM, 128), lambda i: (i, 0)),
            col(_BM), col(_BM),
            full(128, 512),
        ],
        out_specs=[pl.BlockSpec((_BM, 512), lambda i: (i, 0))] * 2,
        out_shape=[jax.ShapeDtypeStruct((N, 512), f32)] * 2,
    )(x, dga, dgb, W1)

    z512 = jnp.zeros((181 * 512,), f32)
    z256 = jnp.zeros((361 * 256,), f32)
    agg1 = _agg512(h1s.reshape(N * 512), srcg, dstg,
                   z512).reshape(N_PAD2, 512)

    h2, h2s = pl.pallas_call(
        _tc_stage2,
        grid=(nblk,),
        in_specs=[
            pl.BlockSpec((_BM, 512), lambda i: (i, 0)),
            pl.BlockSpec((_BM, 512), lambda i: (i, 0)),
            col(_BM), col(_BM),
            full(512, 256), full(1, 512),
        ],
        out_specs=[pl.BlockSpec((_BM, 256), lambda i: (i, 0))] * 2,
        out_shape=[jax.ShapeDtypeStruct((N, 256), f32)] * 2,
    )(agg1, h1, dga, dgb, W2, b1.reshape(1, 512))

    agg2 = _agg256(h2s.reshape(N * 256), srcg, dstg,
                   z256).reshape(N_PAD2, 256)

    out, embs = pl.pallas_call(
        _tc_stage3,
        grid=(nblk,),
        in_specs=[
            pl.BlockSpec((_BM, 256), lambda i: (i, 0)),
            pl.BlockSpec((_BM, 256), lambda i: (i, 0)),
            col(_BM), col(_BM),
            full(1, 256), col(_BM),
            full(256, 128), full(1, 128), full(128, 32), full(1, 32),
            full(32, 10), full(1, 10),
        ],
        out_specs=[
            pl.BlockSpec((G, 10), lambda i: (0, 0)),
            pl.BlockSpec((G, 256), lambda i: (0, 0)),
        ],
        out_shape=[
            jax.ShapeDtypeStruct((G, 10), f32),
            jax.ShapeDtypeStruct((G, 256), f32),
        ],
    )(agg2, h2, dga, dgb, b2.reshape(1, 256), batch.reshape(N, 1),
      fW1, fb1.reshape(1, 128), fW2, fb2.reshape(1, 32),
      fW3, fb3.reshape(1, 10))

    return (out, embs)


# speedup vs baseline: 3.3311x; 1.1904x over previous
import functools

import jax
import jax.numpy as jnp
from jax import lax
from jax.experimental import pallas as pl
from jax.experimental.pallas import tpu as pltpu
from jax.experimental.pallas import tpu_sc as plsc

N = 10000
E = 320000
G = 64
N_PAD = 10240
N_PAD2 = 11520
NEG_INF = float("-inf")

_MESH = plsc.VectorSubcoreMesh(core_axis_name="c", subcore_axis_name="s")
_NTILES = 16
_NCORES = 2
_NW = _NCORES * _NTILES


def _fill(ref, n, val, dtype):
    def body(i, _):
        ref[pl.ds(i * 16, 16)] = jnp.full((16,), val, dtype)
        return 0
    lax.fori_loop(0, n // 16, body, 0)


_DEG_K = 80
_EPW = E // _NW


@functools.partial(
    pl.kernel,
    out_type=jax.ShapeDtypeStruct((_NCORES * N_PAD,), jnp.float32),
    mesh=_MESH,
    scratch_types=[
        pltpu.VMEM((_EPW,), jnp.int32),
        pltpu.VMEM((_DEG_K,), jnp.int32),
        pltpu.VMEM((_DEG_K,), jnp.float32),
        pltpu.VMEM((N_PAD // _NTILES,), jnp.float32),
        pltpu.VMEM_SHARED((N_PAD,), jnp.float32),
    ],
)
def _deg_kernel(dstg_hbm, out_hbm, dst_v, idx_v, ones_v, z_v, acc_sh):
    c = lax.axis_index("c")
    s = lax.axis_index("s")
    w = c * _NTILES + s
    stripe = N_PAD // _NTILES

    _fill(z_v, stripe, 0.0, jnp.float32)
    _fill(ones_v, _DEG_K, 1.0, jnp.float32)
    pltpu.sync_copy(z_v, acc_sh.at[pl.ds(s * stripe, stripe)])
    pltpu.sync_copy(dstg_hbm.at[pl.ds(w * _EPW, _EPW)], dst_v)
    plsc.subcore_barrier()

    def chunk(j, _):
        for t in range(_DEG_K // 16):
            idx_v[pl.ds(t * 16, 16)] = dst_v[pl.ds(j * _DEG_K + t * 16, 16)]
        pltpu.sync_copy(ones_v, acc_sh.at[idx_v], add=True)
        return 0
    lax.fori_loop(0, _EPW // _DEG_K, chunk, 0)

    plsc.subcore_barrier()
    pltpu.sync_copy(acc_sh.at[pl.ds(s * stripe, stripe)],
                    out_hbm.at[pl.ds(c * N_PAD + s * stripe, stripe)])


_WE = 1600


def _make_agg(d_feat, rng_rows, n_pass, krows):
    nwin = E // _WE
    accw = (rng_rows + 1) * d_feat

    @functools.partial(
        pl.kernel,
        out_type=jax.ShapeDtypeStruct((N_PAD2 * d_feat,), jnp.float32),
        mesh=_MESH,
        compiler_params=pltpu.CompilerParams(needs_layout_passes=False),
        scratch_types=[
            pltpu.VMEM((2 * _WE,), jnp.int32),
            pltpu.VMEM((2 * _WE,), jnp.int32),
            pltpu.VMEM((_WE + 32,), jnp.int32),
            pltpu.VMEM((krows * d_feat,), jnp.float32),
            pltpu.VMEM((accw,), jnp.float32),
            pltpu.SemaphoreType.DMA,
            pltpu.SemaphoreType.DMA,
            pltpu.SemaphoreType.DMA,
        ],
    )
    def agg(table_hbm, srcg_hbm, dstg_hbm, zeros_hbm, out_hbm,
            sw, dw, pend, gbuf, acc, semg, sema, semb):
        c = lax.axis_index("c")
        s = lax.axis_index("s")
        w = c * _NTILES + s
        lane = lax.iota(jnp.int32, 16)

        def start_win(i):
            off = (i % 2) * _WE
            sl = pl.ds(i * _WE, _WE)

            @pl.when(i % 2 == 0)
            def _():
                pltpu.async_copy(srcg_hbm.at[sl], sw.at[pl.ds(off, _WE)],
                                 sema)
                pltpu.async_copy(dstg_hbm.at[sl], dw.at[pl.ds(off, _WE)],
                                 sema)

            @pl.when(i % 2 == 1)
            def _():
                pltpu.async_copy(srcg_hbm.at[sl], sw.at[pl.ds(off, _WE)],
                                 semb)
                pltpu.async_copy(dstg_hbm.at[sl], dw.at[pl.ds(off, _WE)],
                                 semb)

        def wait_win(i):
            off = (i % 2) * _WE
            sl = pl.ds(i * _WE, _WE)

            @pl.when(i % 2 == 0)
            def _():
                pltpu.make_async_copy(srcg_hbm.at[sl],
                                      sw.at[pl.ds(off, _WE)], sema).wait()
                pltpu.make_async_copy(dstg_hbm.at[sl],
                                      dw.at[pl.ds(off, _WE)], sema).wait()

            @pl.when(i % 2 == 1)
            def _():
                pltpu.make_async_copy(srcg_hbm.at[sl],
                                      sw.at[pl.ds(off, _WE)], semb).wait()
                pltpu.make_async_copy(dstg_hbm.at[sl],
                                      dw.at[pl.ds(off, _WE)], semb).wait()

        def flush(v, lo):
            siv = v & 0x3FFF
            dlv = jnp.minimum((v >> 14) - lo, rng_rows)
            for l in range(16):
                si = siv[l]
                pltpu.async_copy(
                    table_hbm.at[pl.ds(si * d_feat, d_feat)],
                    gbuf.at[pl.ds(l * d_feat, d_feat)], semg)
            for l in range(16):
                si = siv[l]
                pltpu.make_async_copy(
                    table_hbm.at[pl.ds(si * d_feat, d_feat)],
                    gbuf.at[pl.ds(l * d_feat, d_feat)], semg).wait()
            for l in range(16):
                off = dlv[l] * d_feat
                for q in range(d_feat // 16):
                    plsc.addupdate(acc.at[pl.ds(off + q * 16, 16)],
                                   gbuf[pl.ds(l * d_feat + q * 16, 16)])

        for p in range(n_pass):
            lo = (p * _NW + w) * rng_rows

            pltpu.sync_copy(zeros_hbm, acc)

            def grp(i, pnv):
                sv = sw[pl.ds(i * 16, 16)]
                dv = dw[pl.ds(i * 16, 16)]
                m = (dv >= lo) & (dv < lo + rng_rows)
                packed = sv | (dv << 14)
                ps = plsc.cumsum(m.astype(jnp.int32))
                pos = pnv + ps - 1
                plsc.store_scatter(pend, [pos], packed, mask=m)
                return pnv + plsc.cummax(lax.rev(ps, dimensions=(0,)))

            def win_body(i, pnv):
                wait_win(i)

                @pl.when(i + 1 < nwin)
                def _():
                    start_win(i + 1)
                base = (i % 2) * (_WE // 16)

                def grp2(k, pnv):
                    return grp(base + k, pnv)
                pnv = lax.fori_loop(0, _WE // 16, grp2, pnv)

                pn = pnv[0]
                nfull = pn // 16

                def fl(j, _):
                    flush(pend[pl.ds(j * 16, 16)], lo)
                    return 0
                lax.fori_loop(0, nfull, fl, 0)
                pend[pl.ds(0, 16)] = pend[pl.ds(nfull * 16, 16)]
                return pnv - nfull * 16

            start_win(0)
            pnv = lax.fori_loop(0, nwin, win_body,
                                jnp.zeros((16,), jnp.int32))

            pn = pnv[0]
            dummy = jnp.zeros((16,), jnp.int32) + ((lo + rng_rows) << 14)
            plsc.store_scatter(pend, [pn + lane], dummy)
            flush(pend[pl.ds(0, 16)], lo)

            pltpu.sync_copy(acc.at[pl.ds(0, rng_rows * d_feat)],
                            out_hbm.at[pl.ds(lo * d_feat,
                                             rng_rows * d_feat)])

    return agg


_agg512 = _make_agg(512, 180, 2, 16)
_agg256 = _make_agg(256, 360, 1, 16)


_BM = 1000


def _tc_stage1(x_blk, dga_blk, dgb_blk, w1_blk, h1_ref, h1s_ref):
    deg = dga_blk[...] + dgb_blk[...] + 1.0
    dinv = lax.rsqrt(deg)
    h1 = jnp.dot(x_blk[...], w1_blk[...], preferred_element_type=jnp.float32)
    h1_ref[...] = h1
    h1s_ref[...] = h1 * dinv


def _tc_stage2(agg_blk, h1_blk, dga_blk, dgb_blk, w2_blk, b1_blk,
               h2_ref, h2s_ref):
    deg = dga_blk[...] + dgb_blk[...] + 1.0
    dinv = lax.rsqrt(deg)
    v = dinv * agg_blk[...] + h1_blk[...] / deg + b1_blk[...]
    u = jnp.where(v >= 0, v, 0.01 * v)
    h2 = jnp.dot(u, w2_blk[...], preferred_element_type=jnp.float32)
    h2_ref[...] = h2
    h2s_ref[...] = h2 * dinv


def _tc_stage3(agg_blk, h2_blk, dga_blk, dgb_blk, b2_blk, batch_blk,
               fw1_blk, fb1_blk, fw2_blk, fb2_blk, fw3_blk, fb3_blk,
               out_ref, embs_ref):
    i = pl.program_id(0)

    @pl.when(i == 0)
    def _():
        out_ref[...] = jnp.full(out_ref.shape, NEG_INF, jnp.float32)
        embs_ref[...] = jnp.full(embs_ref.shape, NEG_INF, jnp.float32)

    deg = dga_blk[...] + dgb_blk[...] + 1.0
    dinv = lax.rsqrt(deg)
    h = dinv * agg_blk[...] + h2_blk[...] / deg + b2_blk[...]
    y = jnp.dot(h, fw1_blk[...],
                preferred_element_type=jnp.float32) + fb1_blk[...]
    y = jnp.dot(y, fw2_blk[...],
                preferred_element_type=jnp.float32) + fb2_blk[...]
    y = jnp.dot(y, fw3_blk[...],
                preferred_element_type=jnp.float32) + fb3_blk[...]

    b = batch_blk[...]
    for g in range(G):
        m = b == g
        hm = jnp.max(jnp.where(m, h, NEG_INF), axis=0, keepdims=True)
        ym = jnp.max(jnp.where(m, y, NEG_INF), axis=0, keepdims=True)
        embs_ref[g:g + 1, :] = jnp.maximum(embs_ref[g:g + 1, :], hm)
        out_ref[g:g + 1, :] = jnp.maximum(out_ref[g:g + 1, :], ym)


def kernel(x, edge_index, batch, W1, b1, W2, b2, fW1, fb1, fW2, fb2, fW3, fb3):
    f32 = jnp.float32
    nblk = N // _BM

    srcg = edge_index[0]
    dstg = edge_index[1]
    degs = _deg_kernel(dstg)
    dga = degs[:N].reshape(N, 1)
    dgb = degs[N_PAD:N_PAD + N].reshape(N, 1)

    col = lambda bm: pl.BlockSpec((bm, 1), lambda i: (i, 0))
    full = lambda a, b: pl.BlockSpec((a, b), lambda i: (0, 0))

    h1, h1s = pl.pallas_call(
        _tc_stage1,
        grid=(nblk,),
        in_specs=[
            pl.BlockSpec((_BM, 128), lambda i: (i, 0)),
            col(_BM), col(_BM),
            full(128, 512),
        ],
        out_specs=[pl.BlockSpec((_BM, 512), lambda i: (i, 0))] * 2,
        out_shape=[jax.ShapeDtypeStruct((N, 512), f32)] * 2,
    )(x, dga, dgb, W1)

    z512 = jnp.zeros((181 * 512,), f32)
    z256 = jnp.zeros((361 * 256,), f32)
    agg1 = _agg512(h1s.reshape(N * 512), srcg, dstg,
                   z512).reshape(N_PAD2, 512)

    h2, h2s = pl.pallas_call(
        _tc_stage2,
        grid=(nblk,),
        in_specs=[
            pl.BlockSpec((_BM, 512), lambda i: (i, 0)),
            pl.BlockSpec((_BM, 512), lambda i: (i, 0)),
            col(_BM), col(_BM),
            full(512, 256), full(1, 512),
        ],
        out_specs=[pl.BlockSpec((_BM, 256), lambda i: (i, 0))] * 2,
        out_shape=[jax.ShapeDtypeStruct((N, 256), f32)] * 2,
    )(agg1, h1, dga, dgb, W2, b1.reshape(1, 512))

    agg2 = _agg256(h2s.reshape(N * 256), srcg, dstg,
                   z256).reshape(N_PAD2, 256)

    out, embs = pl.pallas_call(
        _tc_stage3,
        grid=(nblk,),
        in_specs=[
            pl.BlockSpec((_BM, 256), lambda i: (i, 0)),
            pl.BlockSpec((_BM, 256), lambda i: (i, 0)),
            col(_BM), col(_BM),
            full(1, 256), col(_BM),
            full(256, 128), full(1, 128), full(128, 32), full(1, 32),
            full(32, 10), full(1, 10),
        ],
        out_specs=[
            pl.BlockSpec((G, 10), lambda i: (0, 0)),
            pl.BlockSpec((G, 256), lambda i: (0, 0)),
        ],
        out_shape=[
            jax.ShapeDtypeStruct((G, 10), f32),
            jax.ShapeDtypeStruct((G, 256), f32),
        ],
    )(agg2, h2, dga, dgb, b2.reshape(1, 256), batch.reshape(N, 1),
      fW1, fb1.reshape(1, 128), fW2, fb2.reshape(1, 32),
      fW3, fb3.reshape(1, 10))

    return (out, embs)


# speedup vs baseline: 3.3708x; 1.0119x over previous
import functools

import jax
import jax.numpy as jnp
from jax import lax
from jax.experimental import pallas as pl
from jax.experimental.pallas import tpu as pltpu
from jax.experimental.pallas import tpu_sc as plsc

N = 10000
E = 320000
G = 64
N_PAD = 10240
N_PAD2 = 11520
NEG_INF = float("-inf")

_MESH = plsc.VectorSubcoreMesh(core_axis_name="c", subcore_axis_name="s")
_NTILES = 16
_NCORES = 2
_NW = _NCORES * _NTILES


def _fill(ref, n, val, dtype):
    def body(i, _):
        ref[pl.ds(i * 16, 16)] = jnp.full((16,), val, dtype)
        return 0
    lax.fori_loop(0, n // 16, body, 0)


_DEG_K = 80
_EPW = E // _NW


@functools.partial(
    pl.kernel,
    out_type=jax.ShapeDtypeStruct((_NCORES * N_PAD,), jnp.float32),
    mesh=_MESH,
    scratch_types=[
        pltpu.VMEM((_EPW,), jnp.int32),
        pltpu.VMEM((_DEG_K,), jnp.int32),
        pltpu.VMEM((_DEG_K,), jnp.float32),
        pltpu.VMEM((N_PAD // _NTILES,), jnp.float32),
        pltpu.VMEM_SHARED((N_PAD,), jnp.float32),
    ],
)
def _deg_kernel(dstg_hbm, out_hbm, dst_v, idx_v, ones_v, z_v, acc_sh):
    c = lax.axis_index("c")
    s = lax.axis_index("s")
    w = c * _NTILES + s
    stripe = N_PAD // _NTILES

    _fill(z_v, stripe, 0.0, jnp.float32)
    _fill(ones_v, _DEG_K, 1.0, jnp.float32)
    pltpu.sync_copy(z_v, acc_sh.at[pl.ds(s * stripe, stripe)])
    pltpu.sync_copy(dstg_hbm.at[pl.ds(w * _EPW, _EPW)], dst_v)
    plsc.subcore_barrier()

    def chunk(j, _):
        for t in range(_DEG_K // 16):
            idx_v[pl.ds(t * 16, 16)] = dst_v[pl.ds(j * _DEG_K + t * 16, 16)]
        pltpu.sync_copy(ones_v, acc_sh.at[idx_v], add=True)
        return 0
    lax.fori_loop(0, _EPW // _DEG_K, chunk, 0)

    plsc.subcore_barrier()
    pltpu.sync_copy(acc_sh.at[pl.ds(s * stripe, stripe)],
                    out_hbm.at[pl.ds(c * N_PAD + s * stripe, stripe)])


_WE = 1600


def _make_agg(d_feat, rng_rows, n_pass, krows):
    nwin = E // _WE
    accw = (rng_rows + 1) * d_feat

    @functools.partial(
        pl.kernel,
        out_type=jax.ShapeDtypeStruct((N_PAD2 * d_feat,), jnp.float32),
        mesh=_MESH,
        compiler_params=pltpu.CompilerParams(needs_layout_passes=False),
        scratch_types=[
            pltpu.VMEM((2 * _WE,), jnp.int32),
            pltpu.VMEM((2 * _WE,), jnp.int32),
            pltpu.VMEM((_WE + 32,), jnp.int32),
            pltpu.VMEM((krows * d_feat,), jnp.float32),
            pltpu.VMEM((accw,), jnp.float32),
            pltpu.SemaphoreType.DMA,
            pltpu.SemaphoreType.DMA,
            pltpu.SemaphoreType.DMA,
        ],
    )
    def agg(table_hbm, srcg_hbm, dstg_hbm, zeros_hbm, out_hbm,
            sw, dw, pend, gbuf, acc, semg, sema, semb):
        c = lax.axis_index("c")
        s = lax.axis_index("s")
        w = c * _NTILES + s
        lane = lax.iota(jnp.int32, 16)

        def start_win(i):
            off = (i % 2) * _WE
            sl = pl.ds(i * _WE, _WE)

            @pl.when(i % 2 == 0)
            def _():
                pltpu.async_copy(srcg_hbm.at[sl], sw.at[pl.ds(off, _WE)],
                                 sema)
                pltpu.async_copy(dstg_hbm.at[sl], dw.at[pl.ds(off, _WE)],
                                 sema)

            @pl.when(i % 2 == 1)
            def _():
                pltpu.async_copy(srcg_hbm.at[sl], sw.at[pl.ds(off, _WE)],
                                 semb)
                pltpu.async_copy(dstg_hbm.at[sl], dw.at[pl.ds(off, _WE)],
                                 semb)

        def wait_win(i):
            off = (i % 2) * _WE
            sl = pl.ds(i * _WE, _WE)

            @pl.when(i % 2 == 0)
            def _():
                pltpu.make_async_copy(srcg_hbm.at[sl],
                                      sw.at[pl.ds(off, _WE)], sema).wait()
                pltpu.make_async_copy(dstg_hbm.at[sl],
                                      dw.at[pl.ds(off, _WE)], sema).wait()

            @pl.when(i % 2 == 1)
            def _():
                pltpu.make_async_copy(srcg_hbm.at[sl],
                                      sw.at[pl.ds(off, _WE)], semb).wait()
                pltpu.make_async_copy(dstg_hbm.at[sl],
                                      dw.at[pl.ds(off, _WE)], semb).wait()

        def flush(v, lo):
            siv = v & 0x3FFF
            dlv = jnp.minimum((v >> 14) - lo, rng_rows)
            for l in range(16):
                si = siv[l]
                pltpu.async_copy(
                    table_hbm.at[pl.ds(si * d_feat, d_feat)],
                    gbuf.at[pl.ds(l * d_feat, d_feat)], semg)
            for l in range(16):
                si = siv[l]
                pltpu.make_async_copy(
                    table_hbm.at[pl.ds(si * d_feat, d_feat)],
                    gbuf.at[pl.ds(l * d_feat, d_feat)], semg).wait()
            for l in range(16):
                off = dlv[l] * d_feat
                for q in range(d_feat // 16):
                    plsc.addupdate(acc.at[pl.ds(off + q * 16, 16)],
                                   gbuf[pl.ds(l * d_feat + q * 16, 16)])

        for p in range(n_pass):
            lo = (p * _NW + w) * rng_rows

            pltpu.sync_copy(zeros_hbm, acc)

            def grp(i, pnv):
                sv = sw[pl.ds(i * 16, 16)]
                dv = dw[pl.ds(i * 16, 16)]
                m = (dv >= lo) & (dv < lo + rng_rows)
                packed = sv | (dv << 14)
                ps = plsc.cumsum(m.astype(jnp.int32))
                pos = pnv + ps - 1
                plsc.store_scatter(pend, [pos], packed, mask=m)
                return pnv + plsc.all_reduce_population_count(m)

            def win_body(i, pnv):
                wait_win(i)

                @pl.when(i + 1 < nwin)
                def _():
                    start_win(i + 1)
                base = (i % 2) * (_WE // 16)

                def grp2(k, pnv):
                    pnv = grp(base + 2 * k, pnv)
                    return grp(base + 2 * k + 1, pnv)
                pnv = lax.fori_loop(0, _WE // 32, grp2, pnv)

                pn = pnv[0]
                nfull = pn // 16

                def fl(j, _):
                    flush(pend[pl.ds(j * 16, 16)], lo)
                    return 0
                lax.fori_loop(0, nfull, fl, 0)
                pend[pl.ds(0, 16)] = pend[pl.ds(nfull * 16, 16)]
                return pnv - nfull * 16

            start_win(0)
            pnv = lax.fori_loop(0, nwin, win_body,
                                jnp.zeros((16,), jnp.int32))

            pn = pnv[0]
            dummy = jnp.zeros((16,), jnp.int32) + ((lo + rng_rows) << 14)
            plsc.store_scatter(pend, [pn + lane], dummy)
            flush(pend[pl.ds(0, 16)], lo)

            pltpu.sync_copy(acc.at[pl.ds(0, rng_rows * d_feat)],
                            out_hbm.at[pl.ds(lo * d_feat,
                                             rng_rows * d_feat)])

    return agg


_agg512 = _make_agg(512, 180, 2, 16)
_agg256 = _make_agg(256, 360, 1, 16)


_BM = 1000


def _tc_stage1(x_blk, dga_blk, dgb_blk, w1_blk, h1_ref, h1s_ref):
    deg = dga_blk[...] + dgb_blk[...] + 1.0
    dinv = lax.rsqrt(deg)
    h1 = jnp.dot(x_blk[...], w1_blk[...], preferred_element_type=jnp.float32)
    h1_ref[...] = h1
    h1s_ref[...] = h1 * dinv


def _tc_stage2(agg_blk, h1_blk, dga_blk, dgb_blk, w2_blk, b1_blk,
               h2_ref, h2s_ref):
    deg = dga_blk[...] + dgb_blk[...] + 1.0
    dinv = lax.rsqrt(deg)
    v = dinv * agg_blk[...] + h1_blk[...] / deg + b1_blk[...]
    u = jnp.where(v >= 0, v, 0.01 * v)
    h2 = jnp.dot(u, w2_blk[...], preferred_element_type=jnp.float32)
    h2_ref[...] = h2
    h2s_ref[...] = h2 * dinv


def _tc_stage3(agg_blk, h2_blk, dga_blk, dgb_blk, b2_blk, batch_blk,
               fw1_blk, fb1_blk, fw2_blk, fb2_blk, fw3_blk, fb3_blk,
               out_ref, embs_ref):
    i = pl.program_id(0)

    @pl.when(i == 0)
    def _():
        out_ref[...] = jnp.full(out_ref.shape, NEG_INF, jnp.float32)
        embs_ref[...] = jnp.full(embs_ref.shape, NEG_INF, jnp.float32)

    deg = dga_blk[...] + dgb_blk[...] + 1.0
    dinv = lax.rsqrt(deg)
    h = dinv * agg_blk[...] + h2_blk[...] / deg + b2_blk[...]
    y = jnp.dot(h, fw1_blk[...],
                preferred_element_type=jnp.float32) + fb1_blk[...]
    y = jnp.dot(y, fw2_blk[...],
                preferred_element_type=jnp.float32) + fb2_blk[...]
    y = jnp.dot(y, fw3_blk[...],
                preferred_element_type=jnp.float32) + fb3_blk[...]

    b = batch_blk[...]
    for g in range(G):
        m = b == g
        hm = jnp.max(jnp.where(m, h, NEG_INF), axis=0, keepdims=True)
        ym = jnp.max(jnp.where(m, y, NEG_INF), axis=0, keepdims=True)
        embs_ref[g:g + 1, :] = jnp.maximum(embs_ref[g:g + 1, :], hm)
        out_ref[g:g + 1, :] = jnp.maximum(out_ref[g:g + 1, :], ym)


def kernel(x, edge_index, batch, W1, b1, W2, b2, fW1, fb1, fW2, fb2, fW3, fb3):
    f32 = jnp.float32
    nblk = N // _BM

    srcg = edge_index[0]
    dstg = edge_index[1]
    degs = _deg_kernel(dstg)
    dga = degs[:N].reshape(N, 1)
    dgb = degs[N_PAD:N_PAD + N].reshape(N, 1)

    col = lambda bm: pl.BlockSpec((bm, 1), lambda i: (i, 0))
    full = lambda a, b: pl.BlockSpec((a, b), lambda i: (0, 0))

    h1, h1s = pl.pallas_call(
        _tc_stage1,
        grid=(nblk,),
        in_specs=[
            pl.BlockSpec((_BM, 128), lambda i: (i, 0)),
            col(_BM), col(_BM),
            full(128, 512),
        ],
        out_specs=[pl.BlockSpec((_BM, 512), lambda i: (i, 0))] * 2,
        out_shape=[jax.ShapeDtypeStruct((N, 512), f32)] * 2,
    )(x, dga, dgb, W1)

    z512 = jnp.zeros((181 * 512,), f32)
    z256 = jnp.zeros((361 * 256,), f32)
    agg1 = _agg512(h1s.reshape(N * 512), srcg, dstg,
                   z512).reshape(N_PAD2, 512)

    h2, h2s = pl.pallas_call(
        _tc_stage2,
        grid=(nblk,),
        in_specs=[
            pl.BlockSpec((_BM, 512), lambda i: (i, 0)),
            pl.BlockSpec((_BM, 512), lambda i: (i, 0)),
            col(_BM), col(_BM),
            full(512, 256), full(1, 512),
        ],
        out_specs=[pl.BlockSpec((_BM, 256), lambda i: (i, 0))] * 2,
        out_shape=[jax.ShapeDtypeStruct((N, 256), f32)] * 2,
    )(agg1, h1, dga, dgb, W2, b1.reshape(1, 512))

    agg2 = _agg256(h2s.reshape(N * 256), srcg, dstg,
                   z256).reshape(N_PAD2, 256)

    out, embs = pl.pallas_call(
        _tc_stage3,
        grid=(nblk,),
        in_specs=[
            pl.BlockSpec((_BM, 256), lambda i: (i, 0)),
            pl.BlockSpec((_BM, 256), lambda i: (i, 0)),
            col(_BM), col(_BM),
            full(1, 256), col(_BM),
            full(256, 128), full(1, 128), full(128, 32), full(1, 32),
            full(32, 10), full(1, 10),
        ],
        out_specs=[
            pl.BlockSpec((G, 10), lambda i: (0, 0)),
            pl.BlockSpec((G, 256), lambda i: (0, 0)),
        ],
        out_shape=[
            jax.ShapeDtypeStruct((G, 10), f32),
            jax.ShapeDtypeStruct((G, 256), f32),
        ],
    )(agg2, h2, dga, dgb, b2.reshape(1, 256), batch.reshape(N, 1),
      fW1, fb1.reshape(1, 128), fW2, fb2.reshape(1, 32),
      fW3, fb3.reshape(1, 10))

    return (out, embs)


# speedup vs baseline: 3.9871x; 1.1828x over previous
import functools

import jax
import jax.numpy as jnp
from jax import lax
from jax.experimental import pallas as pl
from jax.experimental.pallas import tpu as pltpu
from jax.experimental.pallas import tpu_sc as plsc

N = 10000
E = 320000
G = 64
N_PAD = 10240
N_PAD2 = 11520
NEG_INF = float("-inf")

_MESH = plsc.VectorSubcoreMesh(core_axis_name="c", subcore_axis_name="s")
_NTILES = 16
_NCORES = 2
_NW = _NCORES * _NTILES


def _fill(ref, n, val, dtype):
    def body(i, _):
        ref[pl.ds(i * 16, 16)] = jnp.full((16,), val, dtype)
        return 0
    lax.fori_loop(0, n // 16, body, 0)


_DEG_K = 80
_EPW = E // _NW


@functools.partial(
    pl.kernel,
    out_type=jax.ShapeDtypeStruct((_NCORES * N_PAD,), jnp.float32),
    mesh=_MESH,
    scratch_types=[
        pltpu.VMEM((_EPW,), jnp.int32),
        pltpu.VMEM((_DEG_K,), jnp.int32),
        pltpu.VMEM((_DEG_K,), jnp.float32),
        pltpu.VMEM((N_PAD // _NTILES,), jnp.float32),
        pltpu.VMEM_SHARED((N_PAD,), jnp.float32),
    ],
)
def _deg_kernel(dstg_hbm, out_hbm, dst_v, idx_v, ones_v, z_v, acc_sh):
    c = lax.axis_index("c")
    s = lax.axis_index("s")
    w = c * _NTILES + s
    stripe = N_PAD // _NTILES

    _fill(z_v, stripe, 0.0, jnp.float32)
    _fill(ones_v, _DEG_K, 1.0, jnp.float32)
    pltpu.sync_copy(z_v, acc_sh.at[pl.ds(s * stripe, stripe)])
    pltpu.sync_copy(dstg_hbm.at[pl.ds(w * _EPW, _EPW)], dst_v)
    plsc.subcore_barrier()

    def chunk(j, _):
        for t in range(_DEG_K // 16):
            idx_v[pl.ds(t * 16, 16)] = dst_v[pl.ds(j * _DEG_K + t * 16, 16)]
        pltpu.sync_copy(ones_v, acc_sh.at[idx_v], add=True)
        return 0
    lax.fori_loop(0, _EPW // _DEG_K, chunk, 0)

    plsc.subcore_barrier()
    pltpu.sync_copy(acc_sh.at[pl.ds(s * stripe, stripe)],
                    out_hbm.at[pl.ds(c * N_PAD + s * stripe, stripe)])


_WE = 1600


def _make_agg(d_feat, rng_rows, n_pass, krows):
    nwin = E // _WE
    accw = (rng_rows + 1) * d_feat

    @functools.partial(
        pl.kernel,
        out_type=jax.ShapeDtypeStruct((N_PAD2 * d_feat,), jnp.float32),
        mesh=_MESH,
        compiler_params=pltpu.CompilerParams(needs_layout_passes=False),
        scratch_types=[
            pltpu.VMEM((2 * _WE,), jnp.int32),
            pltpu.VMEM((2 * _WE,), jnp.int32),
            pltpu.VMEM((_WE + 32,), jnp.int32),
            pltpu.VMEM((2 * krows * d_feat,), jnp.float32),
            pltpu.VMEM((accw,), jnp.float32),
            pltpu.SemaphoreType.DMA,
            pltpu.SemaphoreType.DMA,
            pltpu.SemaphoreType.DMA,
        ],
    )
    def agg(table_hbm, srcg_hbm, dstg_hbm, zeros_hbm, out_hbm,
            sw, dw, pend, gbuf, acc, semg, sema, semb):
        c = lax.axis_index("c")
        s = lax.axis_index("s")
        w = c * _NTILES + s
        lane = lax.iota(jnp.int32, 16)

        def start_win(i):
            off = (i % 2) * _WE
            sl = pl.ds(i * _WE, _WE)

            @pl.when(i % 2 == 0)
            def _():
                pltpu.async_copy(srcg_hbm.at[sl], sw.at[pl.ds(off, _WE)],
                                 sema)
                pltpu.async_copy(dstg_hbm.at[sl], dw.at[pl.ds(off, _WE)],
                                 sema)

            @pl.when(i % 2 == 1)
            def _():
                pltpu.async_copy(srcg_hbm.at[sl], sw.at[pl.ds(off, _WE)],
                                 semb)
                pltpu.async_copy(dstg_hbm.at[sl], dw.at[pl.ds(off, _WE)],
                                 semb)

        def wait_win(i):
            off = (i % 2) * _WE
            sl = pl.ds(i * _WE, _WE)

            @pl.when(i % 2 == 0)
            def _():
                pltpu.make_async_copy(srcg_hbm.at[sl],
                                      sw.at[pl.ds(off, _WE)], sema).wait()
                pltpu.make_async_copy(dstg_hbm.at[sl],
                                      dw.at[pl.ds(off, _WE)], sema).wait()

            @pl.when(i % 2 == 1)
            def _():
                pltpu.make_async_copy(srcg_hbm.at[sl],
                                      sw.at[pl.ds(off, _WE)], semb).wait()
                pltpu.make_async_copy(dstg_hbm.at[sl],
                                      dw.at[pl.ds(off, _WE)], semb).wait()

        def fire(v, hoff):
            siv = v & 0x3FFF
            for l in range(16):
                si = siv[l]
                pltpu.async_copy(
                    table_hbm.at[pl.ds(si * d_feat, d_feat)],
                    gbuf.at[pl.ds(hoff + l * d_feat, d_feat)], semg)

        def drain_accum(v, lo, hoff):
            siv = v & 0x3FFF
            dlv = jnp.minimum((v >> 14) - lo, rng_rows)
            for l in range(16):
                si = siv[l]
                pltpu.make_async_copy(
                    table_hbm.at[pl.ds(si * d_feat, d_feat)],
                    gbuf.at[pl.ds(hoff + l * d_feat, d_feat)], semg).wait()
            for l in range(16):
                off = dlv[l] * d_feat
                for q in range(d_feat // 16):
                    plsc.addupdate(
                        acc.at[pl.ds(off + q * 16, 16)],
                        gbuf[pl.ds(hoff + l * d_feat + q * 16, 16)])

        def flush(v, lo):
            fire(v, 0)
            drain_accum(v, lo, 0)

        for p in range(n_pass):
            lo = (p * _NW + w) * rng_rows

            pltpu.sync_copy(zeros_hbm, acc)

            def grp(i, pnv):
                sv = sw[pl.ds(i * 16, 16)]
                dv = dw[pl.ds(i * 16, 16)]
                m = (dv >= lo) & (dv < lo + rng_rows)
                packed = sv | (dv << 14)
                ps = plsc.cumsum(m.astype(jnp.int32))
                pos = pnv + ps - 1
                plsc.store_scatter(pend, [pos], packed, mask=m)
                return pnv + plsc.all_reduce_population_count(m)

            def win_body(i, pnv):
                wait_win(i)

                @pl.when(i + 1 < nwin)
                def _():
                    start_win(i + 1)
                base = (i % 2) * (_WE // 16)

                def grp2(k, pnv):
                    pnv = grp(base + 2 * k, pnv)
                    return grp(base + 2 * k + 1, pnv)
                pnv = lax.fori_loop(0, _WE // 32, grp2, pnv)

                pn = pnv[0]
                nfull = pn // 16

                @pl.when(nfull > 0)
                def _():
                    fire(pend[pl.ds(0, 16)], 0)

                    def fl(j, _):
                        @pl.when(j + 1 < nfull)
                        def _():
                            fire(pend[pl.ds(j * 16 + 16, 16)],
                                 ((j + 1) % 2) * (16 * d_feat))
                        drain_accum(pend[pl.ds(j * 16, 16)], lo,
                                    (j % 2) * (16 * d_feat))
                        return 0
                    lax.fori_loop(0, nfull, fl, 0)
                pend[pl.ds(0, 16)] = pend[pl.ds(nfull * 16, 16)]
                return pnv - nfull * 16

            start_win(0)
            pnv = lax.fori_loop(0, nwin, win_body,
                                jnp.zeros((16,), jnp.int32))

            pn = pnv[0]
            dummy = jnp.zeros((16,), jnp.int32) + ((lo + rng_rows) << 14)
            plsc.store_scatter(pend, [pn + lane], dummy)
            flush(pend[pl.ds(0, 16)], lo)

            pltpu.sync_copy(acc.at[pl.ds(0, rng_rows * d_feat)],
                            out_hbm.at[pl.ds(lo * d_feat,
                                             rng_rows * d_feat)])

    return agg


_agg512 = _make_agg(512, 180, 2, 16)
_agg256 = _make_agg(256, 360, 1, 16)


_BM = 1000


def _tc_stage1(x_blk, dga_blk, dgb_blk, w1_blk, h1_ref, h1s_ref):
    deg = dga_blk[...] + dgb_blk[...] + 1.0
    dinv = lax.rsqrt(deg)
    h1 = jnp.dot(x_blk[...], w1_blk[...], preferred_element_type=jnp.float32)
    h1_ref[...] = h1
    h1s_ref[...] = h1 * dinv


def _tc_stage2(agg_blk, h1_blk, dga_blk, dgb_blk, w2_blk, b1_blk,
               h2_ref, h2s_ref):
    deg = dga_blk[...] + dgb_blk[...] + 1.0
    dinv = lax.rsqrt(deg)
    v = dinv * agg_blk[...] + h1_blk[...] / deg + b1_blk[...]
    u = jnp.where(v >= 0, v, 0.01 * v)
    h2 = jnp.dot(u, w2_blk[...], preferred_element_type=jnp.float32)
    h2_ref[...] = h2
    h2s_ref[...] = h2 * dinv


def _tc_stage3(agg_blk, h2_blk, dga_blk, dgb_blk, b2_blk, batch_blk,
               fw1_blk, fb1_blk, fw2_blk, fb2_blk, fw3_blk, fb3_blk,
               out_ref, embs_ref):
    i = pl.program_id(0)

    @pl.when(i == 0)
    def _():
        out_ref[...] = jnp.full(out_ref.shape, NEG_INF, jnp.float32)
        embs_ref[...] = jnp.full(embs_ref.shape, NEG_INF, jnp.float32)

    deg = dga_blk[...] + dgb_blk[...] + 1.0
    dinv = lax.rsqrt(deg)
    h = dinv * agg_blk[...] + h2_blk[...] / deg + b2_blk[...]
    y = jnp.dot(h, fw1_blk[...],
                preferred_element_type=jnp.float32) + fb1_blk[...]
    y = jnp.dot(y, fw2_blk[...],
                preferred_element_type=jnp.float32) + fb2_blk[...]
    y = jnp.dot(y, fw3_blk[...],
                preferred_element_type=jnp.float32) + fb3_blk[...]

    b = batch_blk[...]
    for g in range(G):
        m = b == g
        hm = jnp.max(jnp.where(m, h, NEG_INF), axis=0, keepdims=True)
        ym = jnp.max(jnp.where(m, y, NEG_INF), axis=0, keepdims=True)
        embs_ref[g:g + 1, :] = jnp.maximum(embs_ref[g:g + 1, :], hm)
        out_ref[g:g + 1, :] = jnp.maximum(out_ref[g:g + 1, :], ym)


def kernel(x, edge_index, batch, W1, b1, W2, b2, fW1, fb1, fW2, fb2, fW3, fb3):
    f32 = jnp.float32
    nblk = N // _BM

    srcg = edge_index[0]
    dstg = edge_index[1]
    degs = _deg_kernel(dstg)
    dga = degs[:N].reshape(N, 1)
    dgb = degs[N_PAD:N_PAD + N].reshape(N, 1)

    col = lambda bm: pl.BlockSpec((bm, 1), lambda i: (i, 0))
    full = lambda a, b: pl.BlockSpec((a, b), lambda i: (0, 0))

    h1, h1s = pl.pallas_call(
        _tc_stage1,
        grid=(nblk,),
        in_specs=[
            pl.BlockSpec((_BM, 128), lambda i: (i, 0)),
            col(_BM), col(_BM),
            full(128, 512),
        ],
        out_specs=[pl.BlockSpec((_BM, 512), lambda i: (i, 0))] * 2,
        out_shape=[jax.ShapeDtypeStruct((N, 512), f32)] * 2,
    )(x, dga, dgb, W1)

    z512 = jnp.zeros((181 * 512,), f32)
    z256 = jnp.zeros((361 * 256,), f32)
    agg1 = _agg512(h1s.reshape(N * 512), srcg, dstg,
                   z512).reshape(N_PAD2, 512)

    h2, h2s = pl.pallas_call(
        _tc_stage2,
        grid=(nblk,),
        in_specs=[
            pl.BlockSpec((_BM, 512), lambda i: (i, 0)),
            pl.BlockSpec((_BM, 512), lambda i: (i, 0)),
            col(_BM), col(_BM),
            full(512, 256), full(1, 512),
        ],
        out_specs=[pl.BlockSpec((_BM, 256), lambda i: (i, 0))] * 2,
        out_shape=[jax.ShapeDtypeStruct((N, 256), f32)] * 2,
    )(agg1, h1, dga, dgb, W2, b1.reshape(1, 512))

    agg2 = _agg256(h2s.reshape(N * 256), srcg, dstg,
                   z256).reshape(N_PAD2, 256)

    out, embs = pl.pallas_call(
        _tc_stage3,
        grid=(nblk,),
        in_specs=[
            pl.BlockSpec((_BM, 256), lambda i: (i, 0)),
            pl.BlockSpec((_BM, 256), lambda i: (i, 0)),
            col(_BM), col(_BM),
            full(1, 256), col(_BM),
            full(256, 128), full(1, 128), full(128, 32), full(1, 32),
            full(32, 10), full(1, 10),
        ],
        out_specs=[
            pl.BlockSpec((G, 10), lambda i: (0, 0)),
            pl.BlockSpec((G, 256), lambda i: (0, 0)),
        ],
        out_shape=[
            jax.ShapeDtypeStruct((G, 10), f32),
            jax.ShapeDtypeStruct((G, 256), f32),
        ],
    )(agg2, h2, dga, dgb, b2.reshape(1, 256), batch.reshape(N, 1),
      fW1, fb1.reshape(1, 128), fW2, fb2.reshape(1, 32),
      fW3, fb3.reshape(1, 10))

    return (out, embs)
